# speedup vs baseline: 4.4052x; 4.4052x over previous
import functools

import jax
import jax.numpy as jnp
from jax import lax
from jax.experimental import pallas as pl

NUM_EXPERTS = 64
TOP_K = 8
N_GROUP = 8
GROUP_SIZE = NUM_EXPERTS // N_GROUP
TOPK_GROUP = 4
ROUTED_SCALING_FACTOR = 2.5

BT = 512


def _gate_block(h_ref, wt_ref, bias_ref, idx_ref, wgt_ref):
    logits = jnp.dot(h_ref[...], wt_ref[...], preferred_element_type=jnp.float32)
    lt = logits.T
    scores = jax.nn.sigmoid(lt)
    sfc = scores + bias_ref[...]

    bt = lt.shape[1]
    rows8 = lax.broadcasted_iota(jnp.int32, (N_GROUP, bt), 0)

    gs_list = []
    for g in range(N_GROUP):
        sub = sfc[g * GROUP_SIZE:(g + 1) * GROUP_SIZE, :]
        m1 = jnp.max(sub, axis=0, keepdims=True)
        first = jnp.min(jnp.where(sub == m1, rows8, N_GROUP), axis=0, keepdims=True)
        m2 = jnp.max(jnp.where(rows8 == first, -jnp.inf, sub), axis=0, keepdims=True)
        gs_list.append(m1 + m2)
    gs = jnp.concatenate(gs_list, axis=0)

    masked_parts = []
    for g in range(N_GROUP):
        row = gs[g:g + 1, :]
        better = (gs > row) | ((gs == row) & (rows8 < g))
        rank = jnp.sum(better.astype(jnp.int32), axis=0, keepdims=True)
        keep = rank < TOPK_GROUP
        sub = sfc[g * GROUP_SIZE:(g + 1) * GROUP_SIZE, :]
        masked_parts.append(jnp.where(keep, sub, 0.0))
    t = jnp.concatenate(masked_parts, axis=0)

    rows64 = lax.broadcasted_iota(jnp.int32, (NUM_EXPERTS, bt), 0)
    idxs, wgts = [], []
    for _ in range(TOP_K):
        m = jnp.max(t, axis=0, keepdims=True)
        idx = jnp.min(jnp.where(t == m, rows64, NUM_EXPERTS), axis=0, keepdims=True)
        sel = rows64 == idx
        w = jnp.max(jnp.where(sel, scores, -jnp.inf), axis=0, keepdims=True)
        t = jnp.where(sel, -jnp.inf, t)
        idxs.append(idx)
        wgts.append(w)
    idx8 = jnp.concatenate(idxs, axis=0)
    w8 = jnp.concatenate(wgts, axis=0)
    w8 = w8 / (jnp.sum(w8, axis=0, keepdims=True) + 1e-20) * ROUTED_SCALING_FACTOR

    idx_ref[...] = idx8.T
    wgt_ref[...] = w8.T


def kernel(hidden_states, weight, e_score_correction_bias):
    hidden = hidden_states.shape[-1]
    hs = hidden_states.reshape(-1, hidden)
    n_tokens = hs.shape[0]
    wt = weight.T
    bias = e_score_correction_bias.reshape(NUM_EXPERTS, 1)

    grid = (n_tokens // BT,)
    idx, wgt = pl.pallas_call(
        _gate_block,
        grid=grid,
        in_specs=[
            pl.BlockSpec((BT, hidden), lambda i: (i, 0)),
            pl.BlockSpec((hidden, NUM_EXPERTS), lambda i: (0, 0)),
            pl.BlockSpec((NUM_EXPERTS, 1), lambda i: (0, 0)),
        ],
        out_specs=[
            pl.BlockSpec((BT, TOP_K), lambda i: (i, 0)),
            pl.BlockSpec((BT, TOP_K), lambda i: (i, 0)),
        ],
        out_shape=[
            jax.ShapeDtypeStruct((n_tokens, TOP_K), jnp.int32),
            jax.ShapeDtypeStruct((n_tokens, TOP_K), jnp.float32),
        ],
    )(hs, wt, bias)
    return idx, wgt


# speedup vs baseline: 4.8046x; 1.0907x over previous
import functools

import jax
import jax.numpy as jnp
from jax import lax
from jax.experimental import pallas as pl

NUM_EXPERTS = 64
TOP_K = 8
N_GROUP = 8
GROUP_SIZE = NUM_EXPERTS // N_GROUP
TOPK_GROUP = 4
ROUTED_SCALING_FACTOR = 2.5

BT = 1024


def _gate_block(h_ref, wt_ref, bias_ref, idx_ref, wgt_ref):
    logits = jnp.dot(h_ref[...], wt_ref[...], preferred_element_type=jnp.float32)
    lt = logits.T
    scores = jax.nn.sigmoid(lt)
    sfc = scores + bias_ref[...]

    bt = lt.shape[1]
    rows8 = lax.broadcasted_iota(jnp.int32, (N_GROUP, bt), 0)

    gs_list = []
    for g in range(N_GROUP):
        sub = sfc[g * GROUP_SIZE:(g + 1) * GROUP_SIZE, :]
        m1 = jnp.max(sub, axis=0, keepdims=True)
        first = jnp.min(jnp.where(sub == m1, rows8, N_GROUP), axis=0, keepdims=True)
        m2 = jnp.max(jnp.where(rows8 == first, -jnp.inf, sub), axis=0, keepdims=True)
        gs_list.append(m1 + m2)
    gs = jnp.concatenate(gs_list, axis=0)

    masked_parts = []
    for g in range(N_GROUP):
        row = gs[g:g + 1, :]
        better = (gs > row) | ((gs == row) & (rows8 < g))
        rank = jnp.sum(better.astype(jnp.int32), axis=0, keepdims=True)
        keep = rank < TOPK_GROUP
        sub = sfc[g * GROUP_SIZE:(g + 1) * GROUP_SIZE, :]
        masked_parts.append(jnp.where(keep, sub, 0.0))
    t = jnp.concatenate(masked_parts, axis=0)

    rows64 = lax.broadcasted_iota(jnp.int32, (NUM_EXPERTS, bt), 0)
    idxs, wgts = [], []
    for _ in range(TOP_K):
        m = jnp.max(t, axis=0, keepdims=True)
        idx = jnp.min(jnp.where(t == m, rows64, NUM_EXPERTS), axis=0, keepdims=True)
        sel = rows64 == idx
        w = jnp.max(jnp.where(sel, scores, -jnp.inf), axis=0, keepdims=True)
        t = jnp.where(sel, -jnp.inf, t)
        idxs.append(idx)
        wgts.append(w)
    idx8 = jnp.concatenate(idxs, axis=0)
    w8 = jnp.concatenate(wgts, axis=0)
    w8 = w8 / (jnp.sum(w8, axis=0, keepdims=True) + 1e-20) * ROUTED_SCALING_FACTOR

    idx_ref[...] = idx8.T
    wgt_ref[...] = w8.T


def kernel(hidden_states, weight, e_score_correction_bias):
    hidden = hidden_states.shape[-1]
    hs = hidden_states.reshape(-1, hidden)
    n_tokens = hs.shape[0]
    wt = weight.T
    bias = e_score_correction_bias.reshape(NUM_EXPERTS, 1)

    grid = (n_tokens // BT,)
    idx, wgt = pl.pallas_call(
        _gate_block,
        grid=grid,
        in_specs=[
            pl.BlockSpec((BT, hidden), lambda i: (i, 0)),
            pl.BlockSpec((hidden, NUM_EXPERTS), lambda i: (0, 0)),
            pl.BlockSpec((NUM_EXPERTS, 1), lambda i: (0, 0)),
        ],
        out_specs=[
            pl.BlockSpec((BT, TOP_K), lambda i: (i, 0)),
            pl.BlockSpec((BT, TOP_K), lambda i: (i, 0)),
        ],
        out_shape=[
            jax.ShapeDtypeStruct((n_tokens, TOP_K), jnp.int32),
            jax.ShapeDtypeStruct((n_tokens, TOP_K), jnp.float32),
        ],
    )(hs, wt, bias)
    return idx, wgt


# speedup vs baseline: 4.8110x; 1.0013x over previous
import functools

import jax
import jax.numpy as jnp
from jax import lax
from jax.experimental import pallas as pl
from jax.experimental.pallas import tpu as pltpu

NUM_EXPERTS = 64
TOP_K = 8
N_GROUP = 8
GROUP_SIZE = NUM_EXPERTS // N_GROUP
TOPK_GROUP = 4
ROUTED_SCALING_FACTOR = 2.5

BT = 1024


def _gate_block(h_ref, wt_ref, bias_ref, idx_ref, wgt_ref):
    logits = jnp.dot(h_ref[...], wt_ref[...], preferred_element_type=jnp.float32)
    lt = logits.T
    scores = jax.nn.sigmoid(lt)
    sfc = scores + bias_ref[...]

    bt = lt.shape[1]
    rows8 = lax.broadcasted_iota(jnp.int32, (N_GROUP, bt), 0)

    gs_list = []
    for g in range(N_GROUP):
        sub = sfc[g * GROUP_SIZE:(g + 1) * GROUP_SIZE, :]
        m1 = jnp.max(sub, axis=0, keepdims=True)
        first = jnp.min(jnp.where(sub == m1, rows8, N_GROUP), axis=0, keepdims=True)
        m2 = jnp.max(jnp.where(rows8 == first, -jnp.inf, sub), axis=0, keepdims=True)
        gs_list.append(m1 + m2)
    gs = jnp.concatenate(gs_list, axis=0)

    masked_parts = []
    for g in range(N_GROUP):
        row = gs[g:g + 1, :]
        better = (gs > row) | ((gs == row) & (rows8 < g))
        rank = jnp.sum(better.astype(jnp.int32), axis=0, keepdims=True)
        keep = rank < TOPK_GROUP
        sub = sfc[g * GROUP_SIZE:(g + 1) * GROUP_SIZE, :]
        masked_parts.append(jnp.where(keep, sub, 0.0))
    t = jnp.concatenate(masked_parts, axis=0)

    rows64 = lax.broadcasted_iota(jnp.int32, (NUM_EXPERTS, bt), 0)
    idxs, wgts = [], []
    for _ in range(TOP_K):
        m = jnp.max(t, axis=0, keepdims=True)
        idx = jnp.min(jnp.where(t == m, rows64, NUM_EXPERTS), axis=0, keepdims=True)
        sel = rows64 == idx
        w = jnp.max(jnp.where(sel, scores, -jnp.inf), axis=0, keepdims=True)
        t = jnp.where(sel, -jnp.inf, t)
        idxs.append(idx)
        wgts.append(w)
    idx8 = jnp.concatenate(idxs, axis=0)
    w8 = jnp.concatenate(wgts, axis=0)
    w8 = w8 / (jnp.sum(w8, axis=0, keepdims=True) + 1e-20) * ROUTED_SCALING_FACTOR

    idx_ref[...] = idx8.T
    wgt_ref[...] = w8.T


def kernel(hidden_states, weight, e_score_correction_bias):
    hidden = hidden_states.shape[-1]
    hs = hidden_states.reshape(-1, hidden)
    n_tokens = hs.shape[0]
    wt = weight.T
    bias = e_score_correction_bias.reshape(NUM_EXPERTS, 1)

    grid = (n_tokens // BT,)
    idx, wgt = pl.pallas_call(
        _gate_block,
        grid=grid,
        in_specs=[
            pl.BlockSpec((BT, hidden), lambda i: (i, 0)),
            pl.BlockSpec((hidden, NUM_EXPERTS), lambda i: (0, 0)),
            pl.BlockSpec((NUM_EXPERTS, 1), lambda i: (0, 0)),
        ],
        out_specs=[
            pl.BlockSpec((BT, TOP_K), lambda i: (i, 0)),
            pl.BlockSpec((BT, TOP_K), lambda i: (i, 0)),
        ],
        out_shape=[
            jax.ShapeDtypeStruct((n_tokens, TOP_K), jnp.int32),
            jax.ShapeDtypeStruct((n_tokens, TOP_K), jnp.float32),
        ],
        compiler_params=pltpu.CompilerParams(
            dimension_semantics=("arbitrary",),
            vmem_limit_bytes=128 * 1024 * 1024,
        ),
    )(hs, wt, bias)
    return idx, wgt
